# baseline (speedup 1.0000x reference)
import functools

import jax
import jax.numpy as jnp
from jax import lax
from jax.experimental import pallas as pl
from jax.experimental.pallas import tpu as pltpu
from jax.experimental.pallas import tpu_sc as plsc

_VOCAB = 100000
_DIM = 512
_SEQ = 2048
_BATCH = 4
_ROWS = _SEQ * _BATCH

_NC = 2
_NS = 16
_NW = _NC * _NS
_LANES = 16

_BPW = _ROWS // _NW
_SPW = _BPW // _BATCH
_WIN = 64
_SEQ_WIN = _WIN // _BATCH
_NCHUNK = _BPW // _WIN


def _pe_table():
    s = jnp.arange(_SEQ, dtype=jnp.float32)[:, None]
    i = jnp.arange(_DIM, dtype=jnp.float32)[None, :]
    k = s / jnp.power(10000.0, 2.0 * i / float(_DIM))
    return jnp.where((jnp.arange(_DIM) % 2) == 1, jnp.cos(k), jnp.sin(k))


def _sc_embed(lut, idx, pe):
    mesh = plsc.VectorSubcoreMesh(
        core_axis_name="c", subcore_axis_name="s", num_cores=_NC
    )

    @functools.partial(
        pl.kernel,
        out_type=jax.ShapeDtypeStruct((_ROWS, _DIM), jnp.float32),
        mesh=mesh,
        scratch_types=[
            pltpu.VMEM((_BPW,), jnp.int32),
            pltpu.VMEM((_SPW, _DIM), jnp.float32),
            pltpu.VMEM((_WIN, _DIM), jnp.float32),
            pltpu.VMEM((_WIN, _DIM), jnp.float32),
            pltpu.SemaphoreType.DMA,
            pltpu.SemaphoreType.DMA,
            pltpu.SemaphoreType.DMA,
            pltpu.SemaphoreType.DMA,
            pltpu.SemaphoreType.DMA,
        ],
    )
    def k(lut_hbm, idx_hbm, pe_hbm, out_hbm,
          idx_v, pe_v, buf0, buf1, gsem0, gsem1, osem0, osem1, psem):
        wid = lax.axis_index("s") * _NC + lax.axis_index("c")
        base = wid * _BPW
        sbase = wid * _SPW

        pe_cp = pltpu.async_copy(pe_hbm.at[pl.ds(sbase, _SPW)], pe_v, psem)
        pltpu.sync_copy(idx_hbm.at[pl.ds(base, _BPW)], idx_v)

        bufs = (buf0, buf1)
        gsems = (gsem0, gsem1)
        osems = (osem0, osem1)
        out_cps = [None, None]

        gather_cps = [None, None]
        gather_cps[0] = pltpu.async_copy(
            lut_hbm.at[idx_v.at[pl.ds(0, _WIN)]], buf0, gsem0
        )
        pe_cp.wait()

        for c in range(_NCHUNK):
            cb = c & 1
            nb = (c + 1) & 1
            if c + 1 < _NCHUNK:
                if out_cps[nb] is not None:
                    out_cps[nb].wait()
                gather_cps[nb] = pltpu.async_copy(
                    lut_hbm.at[idx_v.at[pl.ds((c + 1) * _WIN, _WIN)]],
                    bufs[nb],
                    gsems[nb],
                )
            gather_cps[cb].wait()

            buf = bufs[cb]
            srow0 = c * _SEQ_WIN

            @pl.loop(0, _SEQ_WIN)
            def _(s_loc):
                @pl.loop(0, _DIM, step=_LANES)
                def _(c0):
                    pe_vec = pe_v[srow0 + s_loc, pl.ds(c0, _LANES)]
                    for b in range(_BATCH):
                        plsc.addupdate(
                            buf.at[s_loc * _BATCH + b, pl.ds(c0, _LANES)],
                            pe_vec,
                        )

            out_cps[cb] = pltpu.async_copy(
                buf, out_hbm.at[pl.ds(base + c * _WIN, _WIN)], osems[cb]
            )

        out_cps[0].wait()
        out_cps[1].wait()

    return k(lut, idx, pe)


def kernel(src_input, word_lut):
    idx = src_input[:, :, 0].astype(jnp.int32).reshape(_ROWS)
    pe = _pe_table()
    out = _sc_embed(word_lut, idx, pe)
    return out.reshape(_SEQ, _BATCH, _DIM)


# speedup vs baseline: 1.7349x; 1.7349x over previous
import functools

import jax
import jax.numpy as jnp
import numpy as np
from jax import lax
from jax.experimental import pallas as pl
from jax.experimental.pallas import tpu as pltpu
from jax.experimental.pallas import tpu_sc as plsc

_VOCAB = 100000
_DIM = 512
_SEQ = 2048
_BATCH = 4
_ROWS = _SEQ * _BATCH

_NC = 2
_NS = 16
_NW = _NC * _NS
_LANES = 16

_BPW = _ROWS // _NW
_SPW = _BPW // _BATCH
_WIN = 64
_SEQ_WIN = _WIN // _BATCH
_NCHUNK = _BPW // _WIN


def _pe_table():
    s = np.arange(_SEQ, dtype=np.float32)[:, None]
    i = np.arange(_DIM, dtype=np.float32)[None, :]
    k = (s / np.power(10000.0, (2.0 * i / np.float32(_DIM)).astype(np.float32),
                      dtype=np.float32)).astype(np.float64)
    pe = np.where((np.arange(_DIM) % 2) == 1, np.cos(k), np.sin(k))
    return jnp.asarray(pe.astype(np.float32))


def _sc_embed(lut, idx, pe):
    mesh = plsc.VectorSubcoreMesh(
        core_axis_name="c", subcore_axis_name="s", num_cores=_NC
    )

    @functools.partial(
        pl.kernel,
        out_type=jax.ShapeDtypeStruct((_SEQ, _BATCH, _DIM), jnp.float32),
        mesh=mesh,
        scratch_types=[
            pltpu.VMEM((_BPW,), jnp.int32),
            pltpu.VMEM((_SPW, _DIM), jnp.float32),
            pltpu.VMEM((_WIN, _DIM), jnp.float32),
            pltpu.VMEM((_WIN, _DIM), jnp.float32),
            pltpu.SemaphoreType.DMA,
            pltpu.SemaphoreType.DMA,
            pltpu.SemaphoreType.DMA,
            pltpu.SemaphoreType.DMA,
            pltpu.SemaphoreType.DMA,
        ],
    )
    def k(lut_hbm, idx_hbm, pe_hbm, out_hbm,
          idx_v, pe_v, buf0, buf1, gsem0, gsem1, osem0, osem1, psem):
        wid = lax.axis_index("s") * _NC + lax.axis_index("c")
        base = wid * _BPW
        sbase = wid * _SPW

        pe_cp = pltpu.async_copy(pe_hbm.at[pl.ds(sbase, _SPW)], pe_v, psem)
        pltpu.sync_copy(idx_hbm.at[pl.ds(base, _BPW)], idx_v)

        bufs = (buf0, buf1)
        gsems = (gsem0, gsem1)
        osems = (osem0, osem1)
        out_cps = [None, None]

        gather_cps = [None, None]
        gather_cps[0] = pltpu.async_copy(
            lut_hbm.at[idx_v.at[pl.ds(0, _WIN)]], buf0, gsem0
        )
        pe_cp.wait()

        for c in range(_NCHUNK):
            cb = c & 1
            nb = (c + 1) & 1
            if c + 1 < _NCHUNK:
                if out_cps[nb] is not None:
                    out_cps[nb].wait()
                gather_cps[nb] = pltpu.async_copy(
                    lut_hbm.at[idx_v.at[pl.ds((c + 1) * _WIN, _WIN)]],
                    bufs[nb],
                    gsems[nb],
                )
            gather_cps[cb].wait()

            buf = bufs[cb]
            srow0 = c * _SEQ_WIN

            @pl.loop(0, _SEQ_WIN)
            def _(s_loc):
                @pl.loop(0, _DIM, step=_LANES)
                def _(c0):
                    pe_vec = pe_v[srow0 + s_loc, pl.ds(c0, _LANES)]
                    for b in range(_BATCH):
                        plsc.addupdate(
                            buf.at[s_loc * _BATCH + b, pl.ds(c0, _LANES)],
                            pe_vec,
                        )

            out_cps[cb] = pltpu.async_copy(
                buf,
                out_hbm.reshape(_ROWS, _DIM).at[pl.ds(base + c * _WIN, _WIN)],
                osems[cb],
            )

        out_cps[0].wait()
        out_cps[1].wait()

    return k(lut, idx, pe)


def kernel(src_input, word_lut):
    idx = src_input[:, :, 0].astype(jnp.int32).reshape(_ROWS)
    pe = _pe_table()
    return _sc_embed(word_lut, idx, pe)


# speedup vs baseline: 1.8474x; 1.0648x over previous
import functools

import jax
import jax.numpy as jnp
import numpy as np
from jax import lax
from jax.experimental import pallas as pl
from jax.experimental.pallas import tpu as pltpu
from jax.experimental.pallas import tpu_sc as plsc

_VOCAB = 100000
_DIM = 512
_SEQ = 2048
_BATCH = 4
_ROWS = _SEQ * _BATCH

_NC = 2
_NS = 16
_NW = _NC * _NS
_LANES = 16

_BPW = _ROWS // _NW
_SPW = _BPW // _BATCH
_WIN = 64
_SEQ_WIN = _WIN // _BATCH
_NCHUNK = _BPW // _WIN


def _pe_table():
    s = np.arange(_SEQ, dtype=np.float32)[:, None]
    i = np.arange(_DIM, dtype=np.float32)[None, :]
    k = (s / np.power(10000.0, (2.0 * i / np.float32(_DIM)).astype(np.float32),
                      dtype=np.float32)).astype(np.float64)
    pe = np.where((np.arange(_DIM) % 2) == 1, np.cos(k), np.sin(k))
    return jnp.asarray(pe.astype(np.float32))


def _sc_embed(lut, idx, pe):
    mesh = plsc.VectorSubcoreMesh(
        core_axis_name="c", subcore_axis_name="s", num_cores=_NC
    )

    @functools.partial(
        pl.kernel,
        out_type=jax.ShapeDtypeStruct((_SEQ, _BATCH, _DIM), jnp.float32),
        mesh=mesh,
        scratch_types=[
            pltpu.VMEM((_BPW,), jnp.int32),
            pltpu.VMEM((_SPW, _DIM), jnp.float32),
            pltpu.VMEM((_WIN, _DIM), jnp.float32),
            pltpu.VMEM((_WIN, _DIM), jnp.float32),
            pltpu.SemaphoreType.DMA,
            pltpu.SemaphoreType.DMA,
            pltpu.SemaphoreType.DMA,
            pltpu.SemaphoreType.DMA,
            pltpu.SemaphoreType.DMA,
        ],
    )
    def k(lut_hbm, idx_hbm, pe_hbm, out_hbm,
          idx_v, pe_v, buf0, buf1, gsem0, gsem1, osem0, osem1, psem):
        wid = lax.axis_index("s") * _NC + lax.axis_index("c")
        base = wid * _BPW
        sbase = wid * _SPW

        pe_cp = pltpu.async_copy(pe_hbm.at[pl.ds(sbase, _SPW)], pe_v, psem)
        pltpu.sync_copy(idx_hbm.at[pl.ds(base, _BPW)], idx_v)

        bufs = (buf0, buf1)
        gsems = (gsem0, gsem1)
        osems = (osem0, osem1)
        out_cps = [None, None]

        gather_cps = [None, None]
        gather_cps[0] = pltpu.async_copy(
            lut_hbm.at[idx_v.at[pl.ds(0, _WIN)]], buf0, gsem0
        )
        pe_cp.wait()

        for c in range(_NCHUNK):
            cb = c & 1
            nb = (c + 1) & 1
            if c + 1 < _NCHUNK:
                if out_cps[nb] is not None:
                    out_cps[nb].wait()
                gather_cps[nb] = pltpu.async_copy(
                    lut_hbm.at[idx_v.at[pl.ds((c + 1) * _WIN, _WIN)]],
                    bufs[nb],
                    gsems[nb],
                )
            gather_cps[cb].wait()

            buf = bufs[cb]
            srow0 = c * _SEQ_WIN

            @pl.loop(0, _SEQ_WIN)
            def _(s_loc):
                @plsc.parallel_loop(0, _DIM, step=_LANES, unroll=4)
                def _(c0):
                    pe_vec = pe_v[srow0 + s_loc, pl.ds(c0, _LANES)]
                    for b in range(_BATCH):
                        plsc.addupdate(
                            buf.at[s_loc * _BATCH + b, pl.ds(c0, _LANES)],
                            pe_vec,
                        )

            out_cps[cb] = pltpu.async_copy(
                buf,
                out_hbm.reshape(_ROWS, _DIM).at[pl.ds(base + c * _WIN, _WIN)],
                osems[cb],
            )

        out_cps[0].wait()
        out_cps[1].wait()

    return k(lut, idx, pe)


def kernel(src_input, word_lut):
    idx = src_input[:, :, 0].astype(jnp.int32).reshape(_ROWS)
    pe = _pe_table()
    return _sc_embed(word_lut, idx, pe)


# speedup vs baseline: 1.8644x; 1.0092x over previous
import functools

import jax
import jax.numpy as jnp
import numpy as np
from jax import lax
from jax.experimental import pallas as pl
from jax.experimental.pallas import tpu as pltpu
from jax.experimental.pallas import tpu_sc as plsc

_VOCAB = 100000
_DIM = 512
_SEQ = 2048
_BATCH = 4
_ROWS = _SEQ * _BATCH

_NC = 2
_NS = 16
_NW = _NC * _NS
_LANES = 16

_BPW = _ROWS // _NW
_SPW = _BPW // _BATCH
_WIN = 64
_SEQ_WIN = _WIN // _BATCH
_NCHUNK = _BPW // _WIN


def _pe_table():
    s = np.arange(_SEQ, dtype=np.float32)[:, None]
    i = np.arange(_DIM, dtype=np.float32)[None, :]
    k = (s / np.power(10000.0, (2.0 * i / np.float32(_DIM)).astype(np.float32),
                      dtype=np.float32)).astype(np.float64)
    pe = np.where((np.arange(_DIM) % 2) == 1, np.cos(k), np.sin(k))
    return jnp.asarray(pe.astype(np.float32).reshape(-1))


def _sc_embed(lut, idx, pe):
    mesh = plsc.VectorSubcoreMesh(
        core_axis_name="c", subcore_axis_name="s", num_cores=_NC
    )

    @functools.partial(
        pl.kernel,
        out_type=jax.ShapeDtypeStruct((_SEQ, _BATCH, _DIM), jnp.float32),
        mesh=mesh,
        scratch_types=[
            pltpu.VMEM((_BPW,), jnp.int32),
            pltpu.VMEM((_SPW * _DIM,), jnp.float32),
            pltpu.VMEM((_WIN, _DIM), jnp.float32),
            pltpu.VMEM((_WIN, _DIM), jnp.float32),
            pltpu.SemaphoreType.DMA,
            pltpu.SemaphoreType.DMA,
            pltpu.SemaphoreType.DMA,
            pltpu.SemaphoreType.DMA,
            pltpu.SemaphoreType.DMA,
        ],
    )
    def k(lut_hbm, idx_hbm, pe_hbm, out_hbm,
          idx_v, pe_v, buf0, buf1, gsem0, gsem1, osem0, osem1, psem):
        wid = lax.axis_index("s") * _NC + lax.axis_index("c")
        base = wid * _BPW
        sbase = wid * _SPW

        pe_cp = pltpu.async_copy(
            pe_hbm.at[pl.ds(sbase * _DIM, _SPW * _DIM)], pe_v, psem
        )
        pltpu.sync_copy(idx_hbm.at[pl.ds(base, _BPW)], idx_v)

        bufs = (buf0, buf1)
        gsems = (gsem0, gsem1)
        osems = (osem0, osem1)
        out_cps = [None, None]

        gather_cps = [None, None]
        gather_cps[0] = pltpu.async_copy(
            lut_hbm.at[idx_v.at[pl.ds(0, _WIN)]], buf0, gsem0
        )
        pe_cp.wait()

        for c in range(_NCHUNK):
            cb = c & 1
            nb = (c + 1) & 1
            if c + 1 < _NCHUNK:
                if out_cps[nb] is not None:
                    out_cps[nb].wait()
                gather_cps[nb] = pltpu.async_copy(
                    lut_hbm.at[idx_v.at[pl.ds((c + 1) * _WIN, _WIN)]],
                    bufs[nb],
                    gsems[nb],
                )
            gather_cps[cb].wait()

            buf = bufs[cb]
            srow0 = c * _SEQ_WIN

            @pl.loop(0, _SEQ_WIN)
            def _(s_loc):
                @plsc.parallel_loop(0, _DIM, step=_LANES, unroll=4)
                def _(c0):
                    pe_vec = pe_v[pl.ds((srow0 + s_loc) * _DIM + c0, _LANES)]
                    for b in range(_BATCH):
                        plsc.addupdate(
                            buf.at[s_loc * _BATCH + b, pl.ds(c0, _LANES)],
                            pe_vec,
                        )

            out_cps[cb] = pltpu.async_copy(
                buf,
                out_hbm.reshape(_ROWS, _DIM).at[pl.ds(base + c * _WIN, _WIN)],
                osems[cb],
            )

        out_cps[0].wait()
        out_cps[1].wait()

    return k(lut, idx, pe)


def kernel(src_input, word_lut):
    idx = src_input[:, :, 0].astype(jnp.int32).reshape(_ROWS)
    pe = _pe_table()
    return _sc_embed(word_lut, idx, pe)


# speedup vs baseline: 1.8697x; 1.0029x over previous
import functools

import jax
import jax.numpy as jnp
import numpy as np
from jax import lax
from jax.experimental import pallas as pl
from jax.experimental.pallas import tpu as pltpu
from jax.experimental.pallas import tpu_sc as plsc

_VOCAB = 100000
_DIM = 512
_SEQ = 2048
_BATCH = 4
_ROWS = _SEQ * _BATCH

_NC = 2
_NS = 16
_NW = _NC * _NS
_LANES = 16

_BPW = _ROWS // _NW
_SPW = _BPW // _BATCH
_WIN = 64
_SEQ_WIN = _WIN // _BATCH
_NCHUNK = _BPW // _WIN
_QCHUNKS = _DIM // (2 * _LANES)
_PE_WORDS_PER_ROW = _DIM // 2


def _pe_table_packed():
    s = np.arange(_SEQ, dtype=np.float32)[:, None]
    i = np.arange(_DIM, dtype=np.float32)[None, :]
    k = (s / np.power(10000.0, (2.0 * i / np.float32(_DIM)).astype(np.float32),
                      dtype=np.float32)).astype(np.float64)
    pe = np.where((np.arange(_DIM) % 2) == 1, np.cos(k), np.sin(k))
    pe = np.ascontiguousarray(pe.astype(np.float32))
    b = pe.view(np.uint32)
    bf = ((b + 0x7FFF + ((b >> 16) & 1)) >> 16).astype(np.uint32)
    bf = bf.reshape(_SEQ, _QCHUNKS, 2, _LANES)
    packed = (bf[:, :, 1, :] << 16) | bf[:, :, 0, :]
    return jnp.asarray(packed.view(np.int32).reshape(-1))


def _sc_embed(lut, idx, pe):
    mesh = plsc.VectorSubcoreMesh(
        core_axis_name="c", subcore_axis_name="s", num_cores=_NC
    )

    @functools.partial(
        pl.kernel,
        out_type=jax.ShapeDtypeStruct((_SEQ, _BATCH, _DIM), jnp.float32),
        mesh=mesh,
        compiler_params=pltpu.CompilerParams(needs_layout_passes=False),
        scratch_types=[
            pltpu.VMEM((_BPW,), jnp.int32),
            pltpu.VMEM((_SPW * _PE_WORDS_PER_ROW,), jnp.int32),
            pltpu.VMEM((_WIN, _DIM), jnp.float32),
            pltpu.VMEM((_WIN, _DIM), jnp.float32),
            pltpu.SemaphoreType.DMA,
            pltpu.SemaphoreType.DMA,
            pltpu.SemaphoreType.DMA,
            pltpu.SemaphoreType.DMA,
            pltpu.SemaphoreType.DMA,
        ],
    )
    def k(lut_hbm, idx_hbm, pe_hbm, out_hbm,
          idx_v, pe_v, buf0, buf1, gsem0, gsem1, osem0, osem1, psem):
        wid = lax.axis_index("s") * _NC + lax.axis_index("c")
        base = wid * _BPW
        sbase = wid * _SPW

        pltpu.sync_copy(idx_hbm.at[pl.ds(base, _BPW)], idx_v)

        bufs = (buf0, buf1)
        gsems = (gsem0, gsem1)
        osems = (osem0, osem1)
        out_cps = [None, None]

        gather_cps = [None, None]
        gather_cps[0] = pltpu.async_copy(
            lut_hbm.at[idx_v.at[pl.ds(0, _WIN)]], buf0, gsem0
        )
        pe_cp = pltpu.async_copy(
            pe_hbm.at[pl.ds(sbase * _PE_WORDS_PER_ROW, _SPW * _PE_WORDS_PER_ROW)],
            pe_v,
            psem,
        )

        for c in range(_NCHUNK):
            cb = c & 1
            nb = (c + 1) & 1
            if c + 1 < _NCHUNK:
                if out_cps[nb] is not None:
                    out_cps[nb].wait()
                gather_cps[nb] = pltpu.async_copy(
                    lut_hbm.at[idx_v.at[pl.ds((c + 1) * _WIN, _WIN)]],
                    bufs[nb],
                    gsems[nb],
                )
            if c == 0:
                pe_cp.wait()
            gather_cps[cb].wait()

            buf = bufs[cb]
            pe_row0 = c * _SEQ_WIN

            @pl.loop(0, _SEQ_WIN)
            def _(s_loc):
                pe_base = (pe_row0 + s_loc) * _PE_WORDS_PER_ROW

                @plsc.parallel_loop(0, _PE_WORDS_PER_ROW, step=_LANES, unroll=4)
                def _(q0):
                    packed = pe_v[pl.ds(pe_base + q0, _LANES)]
                    lo = plsc.bitcast(lax.shift_left(packed, 16), jnp.float32)
                    hi = plsc.bitcast(
                        jnp.bitwise_and(packed, jnp.int32(-65536)), jnp.float32
                    )
                    c_lo = q0 * 2
                    for b in range(_BATCH):
                        row = s_loc * _BATCH + b
                        plsc.addupdate(buf.at[row, pl.ds(c_lo, _LANES)], lo)
                        plsc.addupdate(
                            buf.at[row, pl.ds(c_lo + _LANES, _LANES)], hi
                        )

            out_cps[cb] = pltpu.async_copy(
                buf,
                out_hbm.reshape(_ROWS, _DIM).at[pl.ds(base + c * _WIN, _WIN)],
                osems[cb],
            )

        out_cps[0].wait()
        out_cps[1].wait()

    return k(lut, idx, pe)


def kernel(src_input, word_lut):
    idx = jnp.reshape(src_input, (_ROWS,)).astype(jnp.int32)
    pe = _pe_table_packed()
    return _sc_embed(word_lut, idx, pe)


# speedup vs baseline: 1.9451x; 1.0403x over previous
import functools

import jax
import jax.numpy as jnp
import numpy as np
from jax import lax
from jax.experimental import pallas as pl
from jax.experimental.pallas import tpu as pltpu
from jax.experimental.pallas import tpu_sc as plsc

_VOCAB = 100000
_DIM = 512
_SEQ = 2048
_BATCH = 4
_ROWS = _SEQ * _BATCH

_NC = 2
_NS = 16
_NW = _NC * _NS
_LANES = 16

_BPW = _ROWS // _NW
_SPW = _BPW // _BATCH
_WIN = 64
_SEQ_WIN = _WIN // _BATCH
_NCHUNK = _BPW // _WIN
_QCHUNKS = _DIM // (2 * _LANES)
_PE_WORDS_PER_ROW = _DIM // 2


def _pe_table_packed():
    s = np.arange(_SEQ, dtype=np.float32)[:, None]
    i = np.arange(_DIM, dtype=np.float32)[None, :]
    k = (s / np.power(10000.0, (2.0 * i / np.float32(_DIM)).astype(np.float32),
                      dtype=np.float32)).astype(np.float64)
    pe = np.where((np.arange(_DIM) % 2) == 1, np.cos(k), np.sin(k))
    pe = np.ascontiguousarray(pe.astype(np.float32))
    b = pe.view(np.uint32)
    bf = ((b + 0x7FFF + ((b >> 16) & 1)) >> 16).astype(np.uint32)
    bf = bf.reshape(_SEQ, _QCHUNKS, 2, _LANES)
    packed = (bf[:, :, 1, :] << 16) | bf[:, :, 0, :]
    return jnp.asarray(packed.view(np.int32).reshape(-1))


def _sc_embed(lut, idx, pe):
    mesh = plsc.VectorSubcoreMesh(
        core_axis_name="c", subcore_axis_name="s", num_cores=_NC
    )

    @functools.partial(
        pl.kernel,
        out_type=jax.ShapeDtypeStruct((_SEQ, _BATCH, _DIM), jnp.float32),
        mesh=mesh,
        compiler_params=pltpu.CompilerParams(needs_layout_passes=False),
        scratch_types=[
            pltpu.VMEM((_BPW,), jnp.int32),
            pltpu.VMEM((_SPW * _PE_WORDS_PER_ROW,), jnp.int32),
            pltpu.VMEM((_WIN, _DIM), jnp.float32),
            pltpu.VMEM((_WIN, _DIM), jnp.float32),
            pltpu.VMEM((_WIN, _DIM), jnp.float32),
            pltpu.SemaphoreType.DMA,
            pltpu.SemaphoreType.DMA,
            pltpu.SemaphoreType.DMA,
            pltpu.SemaphoreType.DMA,
            pltpu.SemaphoreType.DMA,
            pltpu.SemaphoreType.DMA,
            pltpu.SemaphoreType.DMA,
        ],
    )
    def k(lut_hbm, idx_hbm, pe_hbm, out_hbm,
          idx_v, pe_v, buf0, buf1, buf2,
          gsem0, gsem1, gsem2, osem0, osem1, osem2, psem):
        wid = lax.axis_index("s") * _NC + lax.axis_index("c")
        base = wid * _BPW
        sbase = wid * _SPW

        pltpu.sync_copy(idx_hbm.at[pl.ds(base, _BPW)], idx_v)

        bufs = (buf0, buf1, buf2)
        gsems = (gsem0, gsem1, gsem2)
        osems = (osem0, osem1, osem2)
        out_cps = [None, None, None]

        gather_cps = [None, None, None, None]
        gather_cps[0] = pltpu.async_copy(
            lut_hbm.at[idx_v.at[pl.ds(0, _WIN)]], buf0, gsem0
        )
        gather_cps[1] = pltpu.async_copy(
            lut_hbm.at[idx_v.at[pl.ds(_WIN, _WIN)]], buf1, gsem1
        )
        pe_cp = pltpu.async_copy(
            pe_hbm.at[pl.ds(sbase * _PE_WORDS_PER_ROW, _SPW * _PE_WORDS_PER_ROW)],
            pe_v,
            psem,
        )

        for c in range(_NCHUNK):
            cb = c % 3
            if c + 2 < _NCHUNK:
                fb = (c + 2) % 3
                if out_cps[fb] is not None:
                    out_cps[fb].wait()
                gather_cps[c + 2] = pltpu.async_copy(
                    lut_hbm.at[idx_v.at[pl.ds((c + 2) * _WIN, _WIN)]],
                    bufs[fb],
                    gsems[fb],
                )
            if c == 0:
                pe_cp.wait()
            gather_cps[c].wait()

            buf = bufs[cb]
            pe_row0 = c * _SEQ_WIN

            @pl.loop(0, _SEQ_WIN)
            def _(s_loc):
                pe_base = (pe_row0 + s_loc) * _PE_WORDS_PER_ROW

                @plsc.parallel_loop(0, _PE_WORDS_PER_ROW, step=_LANES, unroll=4)
                def _(q0):
                    packed = pe_v[pl.ds(pe_base + q0, _LANES)]
                    lo = plsc.bitcast(lax.shift_left(packed, 16), jnp.float32)
                    hi = plsc.bitcast(
                        jnp.bitwise_and(packed, jnp.int32(-65536)), jnp.float32
                    )
                    c_lo = q0 * 2
                    for b in range(_BATCH):
                        row = s_loc * _BATCH + b
                        plsc.addupdate(buf.at[row, pl.ds(c_lo, _LANES)], lo)
                        plsc.addupdate(
                            buf.at[row, pl.ds(c_lo + _LANES, _LANES)], hi
                        )

            out_cps[cb] = pltpu.async_copy(
                buf,
                out_hbm.reshape(_ROWS, _DIM).at[pl.ds(base + c * _WIN, _WIN)],
                osems[cb],
            )

        for cp in out_cps:
            if cp is not None:
                cp.wait()

    return k(lut, idx, pe)


def kernel(src_input, word_lut):
    idx = jnp.reshape(src_input, (_ROWS,)).astype(jnp.int32)
    pe = _pe_table_packed()
    return _sc_embed(word_lut, idx, pe)


# speedup vs baseline: 2.0016x; 1.0291x over previous
import functools

import jax
import jax.numpy as jnp
import numpy as np
from jax import lax
from jax.experimental import pallas as pl
from jax.experimental.pallas import tpu as pltpu
from jax.experimental.pallas import tpu_sc as plsc

_VOCAB = 100000
_DIM = 512
_SEQ = 2048
_BATCH = 4
_ROWS = _SEQ * _BATCH

_NC = 2
_NS = 16
_NW = _NC * _NS
_LANES = 16

_BPW = _ROWS // _NW
_SPW = _BPW // _BATCH
_WIN = 32
_SEQ_WIN = _WIN // _BATCH
_NCHUNK = _BPW // _WIN
_NBUF = 6
_QCHUNKS = _DIM // (2 * _LANES)
_PE_WORDS_PER_ROW = _DIM // 2


def _pe_table_packed():
    s = np.arange(_SEQ, dtype=np.float32)[:, None]
    i = np.arange(_DIM, dtype=np.float32)[None, :]
    k = (s / np.power(10000.0, (2.0 * i / np.float32(_DIM)).astype(np.float32),
                      dtype=np.float32)).astype(np.float64)
    pe = np.where((np.arange(_DIM) % 2) == 1, np.cos(k), np.sin(k))
    pe = np.ascontiguousarray(pe.astype(np.float32))
    b = pe.view(np.uint32)
    bf = ((b + 0x7FFF + ((b >> 16) & 1)) >> 16).astype(np.uint32)
    bf = bf.reshape(_SEQ, _QCHUNKS, 2, _LANES)
    packed = (bf[:, :, 1, :] << 16) | bf[:, :, 0, :]
    return jnp.asarray(packed.view(np.int32).reshape(-1))


def _sc_embed(lut, idx, pe):
    mesh = plsc.VectorSubcoreMesh(
        core_axis_name="c", subcore_axis_name="s", num_cores=_NC
    )

    @functools.partial(
        pl.kernel,
        out_type=jax.ShapeDtypeStruct((_SEQ, _BATCH, _DIM), jnp.float32),
        mesh=mesh,
        compiler_params=pltpu.CompilerParams(needs_layout_passes=False),
        scratch_types=(
            [
                pltpu.VMEM((_BPW,), jnp.int32),
                pltpu.VMEM((_SPW * _PE_WORDS_PER_ROW,), jnp.int32),
            ]
            + [pltpu.VMEM((_WIN, _DIM), jnp.float32)] * _NBUF
            + [pltpu.SemaphoreType.DMA] * (2 * _NBUF + 1)
        ),
    )
    def k(lut_hbm, idx_hbm, pe_hbm, out_hbm, *scr):
        idx_v, pe_v = scr[0], scr[1]
        bufs = scr[2 : 2 + _NBUF]
        gsems = scr[2 + _NBUF : 2 + 2 * _NBUF]
        osems = scr[2 + 2 * _NBUF : 2 + 3 * _NBUF]
        psem = scr[2 + 3 * _NBUF]

        wid = lax.axis_index("s") * _NC + lax.axis_index("c")
        base = wid * _BPW
        sbase = wid * _SPW

        out_cps = [None] * _NBUF
        gather_cps = [None] * _NCHUNK

        pltpu.sync_copy(
            idx_hbm.at[pl.ds(base, _WIN)], idx_v.at[pl.ds(0, _WIN)]
        )
        gather_cps[0] = pltpu.async_copy(
            lut_hbm.at[idx_v.at[pl.ds(0, _WIN)]], bufs[0], gsems[0]
        )
        pltpu.sync_copy(
            idx_hbm.at[pl.ds(base + _WIN, _BPW - _WIN)],
            idx_v.at[pl.ds(_WIN, _BPW - _WIN)],
        )
        for c in range(1, _NBUF - 1):
            gather_cps[c] = pltpu.async_copy(
                lut_hbm.at[idx_v.at[pl.ds(c * _WIN, _WIN)]], bufs[c], gsems[c]
            )
        pe_cp = pltpu.async_copy(
            pe_hbm.at[pl.ds(sbase * _PE_WORDS_PER_ROW, _SPW * _PE_WORDS_PER_ROW)],
            pe_v,
            psem,
        )

        for c in range(_NCHUNK):
            cb = c % _NBUF
            if c == 0:
                pe_cp.wait()
            gather_cps[c].wait()

            buf = bufs[cb]
            pe_row0 = c * _SEQ_WIN

            @pl.loop(0, _SEQ_WIN)
            def _(s_loc):
                pe_base = (pe_row0 + s_loc) * _PE_WORDS_PER_ROW

                @plsc.parallel_loop(0, _PE_WORDS_PER_ROW, step=_LANES, unroll=4)
                def _(q0):
                    packed = pe_v[pl.ds(pe_base + q0, _LANES)]
                    lo = plsc.bitcast(lax.shift_left(packed, 16), jnp.float32)
                    hi = plsc.bitcast(
                        jnp.bitwise_and(packed, jnp.int32(-65536)), jnp.float32
                    )
                    c_lo = q0 * 2
                    for b in range(_BATCH):
                        row = s_loc * _BATCH + b
                        plsc.addupdate(buf.at[row, pl.ds(c_lo, _LANES)], lo)
                        plsc.addupdate(
                            buf.at[row, pl.ds(c_lo + _LANES, _LANES)], hi
                        )

            out_cps[cb] = pltpu.async_copy(
                buf,
                out_hbm.reshape(_ROWS, _DIM).at[pl.ds(base + c * _WIN, _WIN)],
                osems[cb],
            )

            nxt = c + _NBUF - 1
            if nxt < _NCHUNK:
                fb = nxt % _NBUF
                if out_cps[fb] is not None:
                    out_cps[fb].wait()
                gather_cps[nxt] = pltpu.async_copy(
                    lut_hbm.at[idx_v.at[pl.ds(nxt * _WIN, _WIN)]],
                    bufs[fb],
                    gsems[fb],
                )

        for cp in out_cps:
            if cp is not None:
                cp.wait()

    return k(lut, idx, pe)


def kernel(src_input, word_lut):
    idx = jnp.reshape(src_input, (_ROWS,)).astype(jnp.int32)
    pe = _pe_table_packed()
    return _sc_embed(word_lut, idx, pe)
